# baseline (speedup 1.0000x reference)
import jax
import jax.numpy as jnp
from jax.experimental import pallas as pl
from jax.experimental.pallas import tpu as pltpu

B, N = 8, 256
GD, ND, ED = 32, 128, 16
NL = 3
EPR = 128 // ED
PACK = N // EPR
NM = N * ED

_f32 = jnp.float32
_bf16 = jnp.bfloat16


def _dot(a, b):
    return jax.lax.dot_general(a, b, (((1,), (0,)), ((), ())),
                               preferred_element_type=_f32)


def _body(*args):
    k = 0

    def take(n):
        nonlocal k
        out = args[k:k + n]
        k += n
        return out

    (A0,) = take(1)
    (V0,) = take(1)
    (u0,) = take(1)
    W = []
    for l in range(1, NL + 1):
        BD = take(l)
        WeVi = take(l)
        WeVj = take(l)
        Weu = take(l)
        (be,) = take(1)
        WvV = take(l)
        (Wve,) = take(1)
        Wvu = take(l)
        (bv,) = take(1)
        Wuu = take(l)
        (WuV,) = take(1)
        (Wue,) = take(1)
        (bu,) = take(1)
        W.append((BD, WeVi, WeVj, Weu, be, WvV, Wve, Wvu, bv,
                  Wuu, WuV, Wue, bu))
    A3, V3, u3 = take(3)
    A0s, A1s, A2s = take(3)

    li = jax.lax.broadcasted_iota(jnp.int32, (128, ED), 0)
    ci = jax.lax.broadcasted_iota(jnp.int32, (128, ED), 1)
    S = (li % ED == ci).astype(_f32)

    Vv = [V0[0]]
    uv = [u0[0]]

    def xg(s, g):
        return (A0s, A1s, A2s)[s][:, g * 128:(g + 1) * 128]

    for l in range(1, NL + 1):
        (BD, WeVi, WeVj, Weu, be, WvV, Wve, Wvu, bv,
         Wuu, WuV, Wue, bu) = W[l - 1]

        P = _dot(Vv[0], WeVi[0][...])
        Q = _dot(Vv[0], WeVj[0][...])
        r = _dot(uv[0], Weu[0][...])
        for s in range(1, l):
            P = P + _dot(Vv[s], WeVi[s][...])
            Q = Q + _dot(Vv[s], WeVj[s][...])
            r = r + _dot(uv[s], Weu[s][...])
        Pr = P + (r + be[...])
        Pt = jnp.tile(Pr, (1, EPR))

        gi = jax.lax.broadcasted_iota(jnp.int32, (PACK, N), 0)
        ri = jax.lax.broadcasted_iota(jnp.int32, (PACK, N), 1)
        qci = jax.lax.broadcasted_iota(jnp.int32, (ED, 128), 0)
        qli = jax.lax.broadcasted_iota(jnp.int32, (ED, 128), 1)
        Qp = jnp.zeros((PACK, 128), _f32)
        for kk in range(EPR):
            Gk = (ri == EPR * gi + kk).astype(_f32)
            Ek = (qli == ED * kk + qci).astype(_f32)
            Qp = Qp + _dot(_dot(Gk, Q), Ek)

        Zsum = jnp.zeros((N, 128), _f32)
        for g in range(PACK):
            if l == 1:
                x0 = A0[0, :, g * 128:(g + 1) * 128].astype(_bf16)
                A0s[:, g * 128:(g + 1) * 128] = x0
            else:
                x0 = xg(0, g)
            Y = _dot(x0, BD[0][...])
            for s in range(1, l):
                Y = Y + _dot(xg(s, g), BD[s][...])
            Z = jnp.maximum(Y + Pt + Qp[g:g + 1, :], 0.0)
            if l == NL:
                A3[0, :, g * 128:(g + 1) * 128] = Z
            else:
                sl = (A1s, A2s)[l - 1]
                sl[:, g * 128:(g + 1) * 128] = Z.astype(_bf16)
            Zsum = Zsum + Z

        agg = _dot(Zsum, S)
        sumA = agg.sum(axis=0, keepdims=True)
        agge = agg * (1.0 / N)

        Vacc = _dot(agge, Wve[...]) + bv[...]
        for s in range(l):
            Vacc = Vacc + _dot(Vv[s], WvV[s][...])
            Vacc = Vacc + _dot(uv[s], Wvu[s][...])
        Vl = jnp.maximum(Vacc, 0.0)

        ua = _dot(Vl.sum(axis=0, keepdims=True) * (1.0 / N), WuV[...])
        ua = ua + _dot(sumA * (1.0 / (N * N)), Wue[...]) + bu[...]
        for s in range(l):
            ua = ua + _dot(uv[s], Wuu[s][...])
        ul = jnp.maximum(ua, 0.0)

        Vv.append(Vl)
        uv.append(ul)

    V3[0] = Vv[NL]
    u3[0] = uv[NL]


def kernel(u, V, A, params):
    Ap = A.reshape(B, N, NM)
    V0 = V.astype(_f32)
    u0 = u.astype(_f32).reshape(B, 1, GD)

    eye8 = jnp.eye(EPR, dtype=_f32)
    weights = []
    for i in range(NL):
        l = i + 1
        We, be, Wv, bv, Wu, bu = params[6 * i: 6 * (i + 1)]
        ie, iv, ig = ED * l, ND * l, GD * l
        weights += [jnp.kron(eye8, We[ED * s: ED * (s + 1), :])
                    .astype(_bf16) for s in range(l)]
        weights += [We[ie + ND * s: ie + ND * (s + 1)] for s in range(l)]
        weights += [We[ie + iv + ND * s: ie + iv + ND * (s + 1)]
                    for s in range(l)]
        weights += [We[ie + 2 * iv + GD * s: ie + 2 * iv + GD * (s + 1)]
                    for s in range(l)]
        weights += [be[None]]
        weights += [Wv[ND * s: ND * (s + 1)] for s in range(l)]
        weights += [Wv[iv: iv + ED]]
        weights += [Wv[iv + ED + GD * s: iv + ED + GD * (s + 1)]
                    for s in range(l)]
        weights += [bv[None]]
        weights += [Wu[GD * s: GD * (s + 1)] for s in range(l)]
        weights += [Wu[ig: ig + ND]]
        weights += [Wu[ig + ND: ig + ND + ED]]
        weights += [bu[None]]

    in_specs = (
        [pl.BlockSpec((1, N, NM), lambda b: (b, 0, 0)),
         pl.BlockSpec((1, N, ND), lambda b: (b, 0, 0)),
         pl.BlockSpec((1, 1, GD), lambda b: (b, 0, 0))]
        + [pl.BlockSpec(w.shape, lambda b, nd=w.ndim: (0,) * nd)
           for w in weights]
    )
    out_specs = [
        pl.BlockSpec((1, N, NM), lambda b: (b, 0, 0)),
        pl.BlockSpec((1, N, ND), lambda b: (b, 0, 0)),
        pl.BlockSpec((1, 1, GD), lambda b: (b, 0, 0)),
    ]
    out_shape = [
        jax.ShapeDtypeStruct((B, N, NM), _f32),
        jax.ShapeDtypeStruct((B, N, ND), _f32),
        jax.ShapeDtypeStruct((B, 1, GD), _f32),
    ]
    scratch = [
        pltpu.VMEM((N, NM), _bf16),
        pltpu.VMEM((N, NM), _bf16),
        pltpu.VMEM((N, NM), _bf16),
    ]

    A3, V3, u3 = pl.pallas_call(
        _body,
        grid=(B,),
        in_specs=in_specs,
        out_specs=out_specs,
        out_shape=out_shape,
        scratch_shapes=scratch,
        input_output_aliases={0: 0},
        compiler_params=pltpu.CompilerParams(
            dimension_semantics=("parallel",)),
    )(Ap, V0, u0, *weights)

    return (u3.reshape(B, GD), V3, A3.reshape(B, N, N, ED))


# speedup vs baseline: 1.0992x; 1.0992x over previous
import jax
import jax.numpy as jnp
from jax.experimental import pallas as pl
from jax.experimental.pallas import tpu as pltpu

B, N = 8, 256
GD, ND, ED = 32, 128, 16
NL = 3
EPR = 128 // ED
PACK = N // EPR
NM = N * ED

_f32 = jnp.float32
_bf16 = jnp.bfloat16


def _dot(a, b):
    return jax.lax.dot_general(a, b, (((1,), (0,)), ((), ())),
                               preferred_element_type=_f32)


def _wspec():
    spec = []
    for l in range(1, NL + 1):
        for s in range(l):
            spec.append(("WeVi", l, s, ND, ED))
        for s in range(l):
            spec.append(("WeVj", l, s, ND, ED))
        for s in range(l):
            spec.append(("Weu", l, s, GD, ED))
        spec.append(("be", l, 0, 1, ED))
        for s in range(l):
            spec.append(("WvV", l, s, ND, ND))
        spec.append(("Wve", l, 0, ED, ND))
        for s in range(l):
            spec.append(("Wvu", l, s, GD, ND))
        spec.append(("bv", l, 0, 1, ND))
        for s in range(l):
            spec.append(("Wuu", l, s, GD, GD))
        spec.append(("WuV", l, 0, ND, GD))
        spec.append(("Wue", l, 0, ED, GD))
        spec.append(("bu", l, 0, 1, GD))
    table = {}
    off = 0
    for name, l, s, rows, cols in spec:
        table[(name, l, s)] = (off, rows, cols)
        off += -(-rows // 8) * 8
    return spec, table, off


_WSPEC, _WTAB, _WROWS = _wspec()
_BDIDX = {}
for _l in range(1, NL + 1):
    for _s in range(_l):
        _BDIDX[(_l, _s)] = len(_BDIDX)


def _body(*args):
    k = 0

    def take(n):
        nonlocal k
        out = args[k:k + n]
        k += n
        return out

    (A0,) = take(1)
    (V0,) = take(1)
    (u0,) = take(1)
    (BDc,) = take(1)
    (Wc,) = take(1)
    A3, V3, u3 = take(3)
    A1s, A2s = take(2)

    def wf(name, l, s=0):
        o, rows, cols = _WTAB[(name, l, s)]
        return Wc[o:o + rows, :cols]

    def bd(l, s):
        o = 128 * _BDIDX[(l, s)]
        return BDc[o:o + 128, :]

    li = jax.lax.broadcasted_iota(jnp.int32, (128, ED), 0)
    ci = jax.lax.broadcasted_iota(jnp.int32, (128, ED), 1)
    S = (li % ED == ci).astype(_f32)

    Vv = [V0[0]]
    uv = [u0[0]]

    def xg(s, g):
        if s == 0:
            return A0[0, :, g * 128:(g + 1) * 128].astype(_bf16)
        return (A1s, A2s)[s - 1][:, g * 128:(g + 1) * 128]

    for l in range(1, NL + 1):
        P = _dot(Vv[0], wf("WeVi", l, 0))
        Q = _dot(Vv[0], wf("WeVj", l, 0))
        r = _dot(uv[0], wf("Weu", l, 0))
        for s in range(1, l):
            P = P + _dot(Vv[s], wf("WeVi", l, s))
            Q = Q + _dot(Vv[s], wf("WeVj", l, s))
            r = r + _dot(uv[s], wf("Weu", l, s))
        Pr = P + (r + wf("be", l))
        Pt = jnp.tile(Pr, (1, EPR))

        gi = jax.lax.broadcasted_iota(jnp.int32, (PACK, N), 0)
        ri = jax.lax.broadcasted_iota(jnp.int32, (PACK, N), 1)
        qci = jax.lax.broadcasted_iota(jnp.int32, (ED, 128), 0)
        qli = jax.lax.broadcasted_iota(jnp.int32, (ED, 128), 1)
        Qp = jnp.zeros((PACK, 128), _f32)
        for kk in range(EPR):
            Gk = (ri == EPR * gi + kk).astype(_f32)
            Ek = (qli == ED * kk + qci).astype(_f32)
            Qp = Qp + _dot(_dot(Gk, Q), Ek)

        Zsum = jnp.zeros((N, 128), _f32)
        for g in range(PACK):
            Y = _dot(xg(0, g), bd(l, 0))
            for s in range(1, l):
                Y = Y + _dot(xg(s, g), bd(l, s))
            Z = jnp.maximum(Y + Pt + Qp[g:g + 1, :], 0.0)
            if l == NL:
                A3[0, :, g * 128:(g + 1) * 128] = Z
            else:
                sl = (A1s, A2s)[l - 1]
                sl[:, g * 128:(g + 1) * 128] = Z.astype(_bf16)
            Zsum = Zsum + Z

        agg = _dot(Zsum, S)
        sumA = agg.sum(axis=0, keepdims=True)
        agge = agg * (1.0 / N)

        Vacc = _dot(agge, wf("Wve", l)) + wf("bv", l)
        for s in range(l):
            Vacc = Vacc + _dot(Vv[s], wf("WvV", l, s))
            Vacc = Vacc + _dot(uv[s], wf("Wvu", l, s))
        Vl = jnp.maximum(Vacc, 0.0)

        ua = _dot(Vl.sum(axis=0, keepdims=True) * (1.0 / N), wf("WuV", l))
        ua = ua + _dot(sumA * (1.0 / (N * N)), wf("Wue", l)) + wf("bu", l)
        for s in range(l):
            ua = ua + _dot(uv[s], wf("Wuu", l, s))
        ul = jnp.maximum(ua, 0.0)

        Vv.append(Vl)
        uv.append(ul)

    V3[0] = Vv[NL]
    u3[0] = uv[NL]


def kernel(u, V, A, params):
    Ap = A.reshape(B, N, NM)
    V0 = V.astype(_f32)
    u0 = u.astype(_f32).reshape(B, 1, GD)

    eye8 = jnp.eye(EPR, dtype=_f32)
    pieces = {}
    bds = []
    for i in range(NL):
        l = i + 1
        We, be, Wv, bv, Wu, bu = params[6 * i: 6 * (i + 1)]
        ie, iv, ig = ED * l, ND * l, GD * l
        bds += [jnp.kron(eye8, We[ED * s: ED * (s + 1), :]).astype(_bf16)
                for s in range(l)]
        for s in range(l):
            pieces[("WeVi", l, s)] = We[ie + ND * s: ie + ND * (s + 1)]
            pieces[("WeVj", l, s)] = We[ie + iv + ND * s:
                                        ie + iv + ND * (s + 1)]
            pieces[("Weu", l, s)] = We[ie + 2 * iv + GD * s:
                                       ie + 2 * iv + GD * (s + 1)]
            pieces[("WvV", l, s)] = Wv[ND * s: ND * (s + 1)]
            pieces[("Wvu", l, s)] = Wv[iv + ED + GD * s:
                                       iv + ED + GD * (s + 1)]
            pieces[("Wuu", l, s)] = Wu[GD * s: GD * (s + 1)]
        pieces[("be", l, 0)] = be[None]
        pieces[("Wve", l, 0)] = Wv[iv: iv + ED]
        pieces[("bv", l, 0)] = bv[None]
        pieces[("WuV", l, 0)] = Wu[ig: ig + ND]
        pieces[("Wue", l, 0)] = Wu[ig + ND: ig + ND + ED]
        pieces[("bu", l, 0)] = bu[None]

    parts = []
    for name, l, s, rows, cols in _WSPEC:
        p = pieces[(name, l, s)]
        rpad = -(-rows // 8) * 8 - rows
        parts.append(jnp.pad(p, ((0, rpad), (0, 128 - cols))))
    Wcat = jnp.concatenate(parts, axis=0)
    BDcat = jnp.concatenate(bds, axis=0)

    in_specs = [
        pl.BlockSpec((1, N, NM), lambda b: (b, 0, 0)),
        pl.BlockSpec((1, N, ND), lambda b: (b, 0, 0)),
        pl.BlockSpec((1, 1, GD), lambda b: (b, 0, 0)),
        pl.BlockSpec(BDcat.shape, lambda b: (0, 0)),
        pl.BlockSpec(Wcat.shape, lambda b: (0, 0)),
    ]
    out_specs = [
        pl.BlockSpec((1, N, NM), lambda b: (b, 0, 0)),
        pl.BlockSpec((1, N, ND), lambda b: (b, 0, 0)),
        pl.BlockSpec((1, 1, GD), lambda b: (b, 0, 0)),
    ]
    out_shape = [
        jax.ShapeDtypeStruct((B, N, NM), _f32),
        jax.ShapeDtypeStruct((B, N, ND), _f32),
        jax.ShapeDtypeStruct((B, 1, GD), _f32),
    ]
    scratch = [
        pltpu.VMEM((N, NM), _bf16),
        pltpu.VMEM((N, NM), _bf16),
    ]

    A3, V3, u3 = pl.pallas_call(
        _body,
        grid=(B,),
        in_specs=in_specs,
        out_specs=out_specs,
        out_shape=out_shape,
        scratch_shapes=scratch,
        input_output_aliases={0: 0},
        compiler_params=pltpu.CompilerParams(
            dimension_semantics=("parallel",)),
    )(Ap, V0, u0, BDcat, Wcat)

    return (u3.reshape(B, GD), V3, A3.reshape(B, N, N, ED))
